# baseline (speedup 1.0000x reference)
import functools

import jax
import jax.numpy as jnp
from jax import lax
from jax.experimental import pallas as pl
from jax.experimental.pallas import tpu as pltpu
from jax.experimental.pallas import tpu_sc as plsc

N_B = 16384
N_S = 50
D_EMBED = 64
CHUNK = 128


def _build_gather():
    info = plsc.get_sparse_core_info()
    num_cores, num_subcores = info.num_cores, info.num_subcores
    num_workers = num_cores * num_subcores
    bt_per_w = (N_B // CHUNK) // num_workers
    n_chunks = N_S * bt_per_w

    mesh = plsc.VectorSubcoreMesh(core_axis_name="c", subcore_axis_name="s")

    @functools.partial(
        pl.kernel,
        mesh=mesh,
        out_type=jax.ShapeDtypeStruct((N_S, 8, 128, 8, 128), jnp.float32),
        compiler_params=pltpu.CompilerParams(
            use_tc_tiling_on_sc=False,
            needs_layout_passes=False,
            disable_bounds_checks=True,
        ),
        scratch_types=[
            pltpu.VMEM((N_S, bt_per_w * CHUNK), jnp.int32),
            pltpu.VMEM((2, CHUNK, D_EMBED), jnp.float32),
            pltpu.VMEM((2, 8, 8, CHUNK), jnp.float32),
            pltpu.SemaphoreType.DMA((2,)),
            pltpu.SemaphoreType.DMA((2,)),
        ],
    )
    def gather_kernel(table_hbm, idxT_hbm, out_hbm, idx_v, gbuf, obuf, gsem, ssem):
        wid = lax.axis_index("s") * num_cores + lax.axis_index("c")
        col0 = wid * (bt_per_w * CHUNK)

        pltpu.sync_copy(idxT_hbm.at[:, pl.ds(col0, bt_per_w * CHUNK)], idx_v)

        def fire_gather(c, p):
            s = c // bt_per_w
            q = c - s * bt_per_w
            pltpu.async_copy(
                table_hbm.at[idx_v.at[s, pl.ds(q * CHUNK, CHUNK)]],
                gbuf.at[p],
                gsem.at[p],
            )

        def wait_gather(p):
            pltpu.make_async_copy(
                table_hbm.at[idx_v.at[0, pl.ds(0, CHUNK)]], gbuf.at[p], gsem.at[p]
            ).wait()

        def fire_stores(c, p):
            s = c // bt_per_w
            bt = wid * bt_per_w + (c - s * bt_per_w)
            for dt in range(8):
                pltpu.async_copy(
                    obuf.at[p, dt], out_hbm.at[s, dt, bt], ssem.at[p]
                )

        def wait_stores(p):
            for dt in range(8):
                pltpu.make_async_copy(
                    obuf.at[p, dt], out_hbm.at[0, dt, 0], ssem.at[p]
                ).wait()

        iota16 = lax.iota(jnp.int32, 16)
        bvecs = [iota16 + 16 * bg for bg in range(8)]

        def transpose_chunk(p):
            src = gbuf.at[p]
            for d in range(D_EMBED):
                colv = jnp.zeros((16,), jnp.int32) + d
                for bg in range(8):
                    val = plsc.load_gather(src, [bvecs[bg], colv])
                    obuf[p, d // 8, d % 8, pl.ds(bg * 16, 16)] = val

        fire_gather(0, 0)

        def body(ci, carry):
            for j in range(2):
                c = ci * 2 + j
                p = j
                @pl.when(c < n_chunks - 1)
                def _():
                    fire_gather(c + 1, 1 - p)
                @pl.when(ci > 0)
                def _():
                    wait_stores(p)
                wait_gather(p)
                transpose_chunk(p)
                fire_stores(c, p)
            return carry

        lax.fori_loop(0, n_chunks // 2, body, 0)
        wait_stores(0)
        wait_stores(1)

    return gather_kernel


_gather = _build_gather()


def kernel(pathway_indices, embedding_table):
    idxT = pathway_indices.T.astype(jnp.int32)
    a = _gather(embedding_table, idxT)
    return jnp.transpose(a, (2, 4, 0, 1, 3)).reshape(N_B, N_S, D_EMBED)


# speedup vs baseline: 1.7816x; 1.7816x over previous
import functools

import jax
import jax.numpy as jnp
from jax import lax
from jax.experimental import pallas as pl
from jax.experimental.pallas import tpu as pltpu
from jax.experimental.pallas import tpu_sc as plsc

N_B = 16384
N_S = 50
D_EMBED = 64
CHUNK = 128


def _build_gather():
    info = plsc.get_sparse_core_info()
    num_cores, num_subcores = info.num_cores, info.num_subcores
    num_workers = num_cores * num_subcores
    bt_per_w = (N_B // CHUNK) // num_workers
    n_chunks = N_S * bt_per_w

    mesh = plsc.VectorSubcoreMesh(core_axis_name="c", subcore_axis_name="s")

    @functools.partial(
        pl.kernel,
        mesh=mesh,
        out_type=jax.ShapeDtypeStruct((N_S, 8, 128, 8, 128), jnp.float32),
        compiler_params=pltpu.CompilerParams(
            use_tc_tiling_on_sc=False,
            needs_layout_passes=False,
            disable_bounds_checks=True,
        ),
        scratch_types=[
            pltpu.VMEM((N_S, bt_per_w * CHUNK), jnp.int32),
            pltpu.VMEM((2, CHUNK, D_EMBED), jnp.float32),
            pltpu.VMEM((2, D_EMBED, CHUNK + 1), jnp.float32),
            pltpu.SemaphoreType.DMA((2,)),
            pltpu.SemaphoreType.DMA((2,)),
        ],
    )
    def gather_kernel(table_hbm, idxT_hbm, out_hbm, idx_v, gbuf, obuf, gsem, ssem):
        wid = lax.axis_index("s") * num_cores + lax.axis_index("c")
        col0 = wid * (bt_per_w * CHUNK)

        pltpu.sync_copy(idxT_hbm.at[:, pl.ds(col0, bt_per_w * CHUNK)], idx_v)

        def fire_gather(c, p):
            s = c // bt_per_w
            q = c - s * bt_per_w
            pltpu.async_copy(
                table_hbm.at[idx_v.at[s, pl.ds(q * CHUNK, CHUNK)]],
                gbuf.at[p],
                gsem.at[p],
            )

        def wait_gather(p):
            pltpu.make_async_copy(
                table_hbm.at[idx_v.at[0, pl.ds(0, CHUNK)]],
                gbuf.at[p],
                gsem.at[p],
            ).wait()

        def fire_stores(c, p):
            s = c // bt_per_w
            bt = wid * bt_per_w + (c - s * bt_per_w)
            for dt in range(8):
                pltpu.async_copy(
                    obuf.at[p, pl.ds(dt * 8, 8), pl.ds(0, CHUNK)],
                    out_hbm.at[s, dt, bt],
                    ssem.at[p],
                )

        def wait_stores(p):
            for dt in range(8):
                pltpu.make_async_copy(
                    obuf.at[p, pl.ds(dt * 8, 8), pl.ds(0, CHUNK)],
                    out_hbm.at[0, dt, 0],
                    ssem.at[p],
                ).wait()

        iota16 = lax.iota(jnp.int32, 16)
        bvecs = [iota16 + 16 * bg for bg in range(8)]

        def transpose_chunk(p):
            dst = obuf.at[p]
            for b in range(CHUNK):
                bsplat = jnp.zeros((16,), jnp.int32) + b
                for k in range(4):
                    val = gbuf[p, b, pl.ds(k * 16, 16)]
                    plsc.store_scatter(dst, [bvecs[k], bsplat], val)

        fire_gather(0, 0)

        def body(ci, carry):
            for j in range(2):
                c = ci * 2 + j
                p = j
                @pl.when(c < n_chunks - 1)
                def _():
                    fire_gather(c + 1, 1 - p)
                @pl.when(ci > 0)
                def _():
                    wait_stores(p)
                wait_gather(p)
                transpose_chunk(p)
                fire_stores(c, p)
            return carry

        lax.fori_loop(0, n_chunks // 2, body, 0)
        wait_stores(0)
        wait_stores(1)

    return gather_kernel


_gather = _build_gather()


def kernel(pathway_indices, embedding_table):
    idxT = pathway_indices.T.astype(jnp.int32)
    a = _gather(embedding_table, idxT)
    return jnp.transpose(a, (2, 4, 0, 1, 3)).reshape(N_B, N_S, D_EMBED)
